# baseline (speedup 1.0000x reference)
import functools

import jax
import jax.numpy as jnp
from jax import lax
from jax.experimental import pallas as pl
from jax.experimental.pallas import tpu as pltpu
from jax.experimental.pallas import tpu_sc as plsc

_NMS_THR = 0.3
_MAX_OUT = 320
_SPEC_K = 8
_CH = 128


def _rank_body(s_rows_ref, s_cols_ref, rank_ref):
    nb, npad = s_rows_ref.shape
    nch = npad // _CH
    jcol = lax.broadcasted_iota(jnp.int32, (_CH, 1), 0)
    irow = lax.broadcasted_iota(jnp.int32, (1, npad), 1)
    ones_row = jnp.ones((1, _CH), jnp.float32)
    dn = (((1,), (0,)), ((), ()))
    for b in range(nb):
        s_row = s_rows_ref[b:b + 1, :]

        def jstep(j, acc, b=b, s_row=s_row):
            sj = s_cols_ref[pl.ds(j * _CH, _CH), b:b + 1]
            jidx = jcol + j * _CH
            before = (sj > s_row) | ((sj == s_row) & (jidx < irow))
            cmp = jnp.where(before, 1.0, 0.0)
            return acc + lax.dot_general(ones_row, cmp, dn,
                                         preferred_element_type=jnp.float32)

        rank = lax.fori_loop(0, nch, jstep, jnp.zeros((1, npad), jnp.float32))
        rank_ref[b:b + 1, :] = rank.astype(jnp.int32)


def _permute_body(rank_hbm, x1_hbm, y1_hbm, x2_hbm, y2_hbm,
                  sx1_hbm, sy1_hbm, sx2_hbm, sy2_hbm,
                  r_s, ord_s, i1_s, i2_s, i3_s, i4_s, o1_s, o2_s, o3_s, o4_s):
    nb = rank_hbm.shape[0]
    npad = rank_hbm.shape[1]
    wid = lax.axis_index("s") * 2 + lax.axis_index("c")

    @pl.when(wid < nb)
    def _():
        pltpu.sync_copy(rank_hbm.at[wid], r_s)
        pltpu.sync_copy(x1_hbm.at[wid], i1_s)
        pltpu.sync_copy(y1_hbm.at[wid], i2_s)
        pltpu.sync_copy(x2_hbm.at[wid], i3_s)
        pltpu.sync_copy(y2_hbm.at[wid], i4_s)

        def scat(i, _):
            rv = r_s[pl.ds(i * 16, 16)]
            iv = lax.iota(jnp.int32, 16) + i * 16
            plsc.store_scatter(ord_s, [rv], iv)
            return 0

        lax.fori_loop(0, npad // 16, scat, 0)

        def gath(i, _):
            ov = ord_s[pl.ds(i * 16, 16)]
            o1_s[pl.ds(i * 16, 16)] = plsc.load_gather(i1_s, [ov])
            o2_s[pl.ds(i * 16, 16)] = plsc.load_gather(i2_s, [ov])
            o3_s[pl.ds(i * 16, 16)] = plsc.load_gather(i3_s, [ov])
            o4_s[pl.ds(i * 16, 16)] = plsc.load_gather(i4_s, [ov])
            return 0

        lax.fori_loop(0, npad // 16, gath, 0)

        pltpu.sync_copy(o1_s, sx1_hbm.at[wid])
        pltpu.sync_copy(o2_s, sy1_hbm.at[wid])
        pltpu.sync_copy(o3_s, sx2_hbm.at[wid])
        pltpu.sync_copy(o4_s, sy2_hbm.at[wid])


def _nms_body(x1_ref, y1_ref, x2_ref, y2_ref, keep_ref, s_ref, a_ref, *, n_real):
    n_b = x1_ref.shape[0]
    rows, fcols = x1_ref.shape[1], x1_ref.shape[2]
    col = lax.broadcasted_iota(jnp.int32, (rows, fcols), 1)
    row = lax.broadcasted_iota(jnp.int32, (rows, fcols), 0)
    gpos = col + fcols * row
    for b in range(n_b):
        x1 = x1_ref[b]
        y1 = y1_ref[b]
        x2 = x2_ref[b]
        y2 = y2_ref[b]
        a_ref[b] = jnp.maximum(x2 - x1, 0.0) * jnp.maximum(y2 - y1, 0.0)
        s_ref[b] = jnp.where(gpos < n_real, 1.0, -1.0)

    def red(op, arr):
        return op(op(arr, axis=1, keepdims=True), axis=0, keepdims=True)

    def lane_shift(arr, s):
        z = jnp.zeros((rows, s), jnp.float32)
        return jnp.concatenate([z, arr[:, :fcols - s]], axis=1)

    def chain(b, _):
        s = s_ref[b]
        a = a_ref[b]
        x1 = x1_ref[b]
        y1 = y1_ref[b]
        x2 = x2_ref[b]
        y2 = y2_ref[b]

        amask = s > 0.0
        af = jnp.where(amask, 1.0, 0.0)

        p = af
        sh = 1
        while sh < fcols:
            p = p + lane_shift(p, sh)
            sh *= 2
        row_tot = p[:, fcols - 1:fcols]
        run = row_tot
        excl = jnp.zeros((rows, 1), jnp.float32)
        step = 1
        while step < rows:
            shifted = jnp.concatenate(
                [jnp.zeros((step, 1), jnp.float32), run[:rows - step]], axis=0)
            excl = excl + shifted
            run = run + shifted
            step *= 2
        p_excl = p + excl - af
        tot = excl[rows - 1:rows] + row_tot[rows - 1:rows]

        cands = []
        for g in range(_SPEC_K):
            cand = amask & (p_excl == float(g))
            act = tot > float(g)
            cands.append((cand, act))

        coords = []
        for cand, act in cands:
            cf = jnp.where(cand, 1.0, 0.0)
            coords.append((red(jnp.sum, x1 * cf), red(jnp.sum, y1 * cf),
                           red(jnp.sum, x2 * cf), red(jnp.sum, y2 * cf),
                           red(jnp.sum, a * cf)))

        def pair_iou(i, j):
            ix1, iy1, ix2, iy2, ia = coords[i]
            jx1, jy1, jx2, jy2, ja = coords[j]
            w = jnp.maximum(jnp.minimum(ix2, jx2) - jnp.maximum(ix1, jx1), 0.0)
            h = jnp.maximum(jnp.minimum(iy2, jy2) - jnp.maximum(iy1, jy1), 0.0)
            inter = w * h
            return inter / jnp.maximum(ia + ja - inter, 1e-9)

        commit = [cands[0][1]]
        for g in range(1, _SPEC_K):
            sup = commit[0] & (pair_iou(0, g) > _NMS_THR)
            for h in range(1, g):
                sup = sup | (commit[h] & (pair_iou(h, g) > _NMS_THR))
            commit.append(cands[g][1] & jnp.logical_not(sup))

        supp = None
        commit_mask = None
        for g in range(_SPEC_K):
            gx1, gy1, gx2, gy2, ga = coords[g]
            w = jnp.maximum(jnp.minimum(x2, gx2) - jnp.maximum(x1, gx1), 0.0)
            h = jnp.maximum(jnp.minimum(y2, gy2) - jnp.maximum(y1, gy1), 0.0)
            inter = w * h
            iou = inter / jnp.maximum(ga + a - inter, 1e-9)
            sg = (iou > _NMS_THR) & commit[g]
            cg = cands[g][0] & commit[g]
            supp = sg if supp is None else (supp | sg)
            commit_mask = cg if commit_mask is None else (commit_mask | cg)

        s_ref[b] = jnp.where(commit_mask, -2.0, jnp.where(supp, -1.0, s))
        resolved = jnp.where((supp | commit_mask) & amask, 1.0, 0.0)
        return (tot - red(jnp.sum, resolved))[0, 0]

    def body(carry):
        return tuple(chain(b, carry[b]) for b in range(n_b))

    def cond(carry):
        m = carry[0]
        for b in range(1, n_b):
            m = jnp.maximum(m, carry[b])
        return m > 0.5

    init = tuple(
        red(jnp.sum, jnp.where(s_ref[b] > 0.0, 1.0, 0.0))[0, 0]
        for b in range(n_b))
    lax.while_loop(cond, body, init)
    for b in range(n_b):
        keep_ref[b] = (s_ref[b] == -2.0).astype(jnp.float32)


def _compact_body(rank_hbm, keep_hbm, cls_hbm, x1_hbm, y1_hbm, x2_hbm, y2_hbm,
                  sc_hbm,
                  oc_hbm, o1_hbm, o2_hbm, o3_hbm, o4_hbm, os_hbm,
                  r_s, k_s, c_s, x1_s, y1_s, x2_s, y2_s, s_s,
                  oc_s, o1_s, o2_s, o3_s, o4_s, os_s):
    nb = keep_hbm.shape[0]
    npad = keep_hbm.shape[1]
    wid = lax.axis_index("s") * 2 + lax.axis_index("c")

    @pl.when(wid < nb)
    def _():
        pltpu.sync_copy(rank_hbm.at[wid], r_s)
        pltpu.sync_copy(keep_hbm.at[wid], k_s)
        pltpu.sync_copy(cls_hbm.at[wid], c_s)
        pltpu.sync_copy(x1_hbm.at[wid], x1_s)
        pltpu.sync_copy(y1_hbm.at[wid], y1_s)
        pltpu.sync_copy(x2_hbm.at[wid], x2_s)
        pltpu.sync_copy(y2_hbm.at[wid], y2_s)
        pltpu.sync_copy(sc_hbm.at[wid], s_s)

        outs = (oc_s, o1_s, o2_s, o3_s, o4_s, os_s)
        srcs = (c_s, x1_s, y1_s, x2_s, y2_s, s_s)

        def zero(i, _):
            z = jnp.zeros((16,), jnp.float32)
            for oref in outs:
                oref[pl.ds(i * 16, 16)] = z
            return 0

        lax.fori_loop(0, _MAX_OUT // 16, zero, 0)

        def step(i, base):
            rv = r_s[pl.ds(i * 16, 16)]
            kv = plsc.load_gather(k_s, [rv])
            ci = plsc.cumsum(kv)
            pos = base + ci.astype(jnp.int32) - 1
            msk = (kv > 0.5) & (pos < _MAX_OUT)
            for src, dst in zip(srcs, outs):
                plsc.store_scatter(dst, [pos], src[pl.ds(i * 16, 16)], mask=msk)
            return base + jnp.sum(kv).astype(jnp.int32)

        lax.fori_loop(0, npad // 16, step, jnp.int32(0))

        pltpu.sync_copy(oc_s, oc_hbm.at[wid])
        pltpu.sync_copy(o1_s, o1_hbm.at[wid])
        pltpu.sync_copy(o2_s, o2_hbm.at[wid])
        pltpu.sync_copy(o3_s, o3_hbm.at[wid])
        pltpu.sync_copy(o4_s, o4_hbm.at[wid])
        pltpu.sync_copy(os_s, os_hbm.at[wid])


@jax.jit
def _run_rank(s_rows, s_cols):
    nb, npad = s_rows.shape
    return pl.pallas_call(
        _rank_body,
        out_shape=jax.ShapeDtypeStruct((nb, npad), jnp.int32),
    )(s_rows, s_cols)


@jax.jit
def _run_permute(rank, x1, y1, x2, y2):
    nb, npad = rank.shape
    mesh = plsc.VectorSubcoreMesh(core_axis_name="c", subcore_axis_name="s")
    out_type = [jax.ShapeDtypeStruct((nb, npad), jnp.float32)] * 4
    scratch = [pltpu.VMEM((npad,), jnp.int32)] * 2 + \
              [pltpu.VMEM((npad,), jnp.float32)] * 8
    return pl.kernel(
        _permute_body,
        out_type=out_type,
        mesh=mesh,
        scratch_types=scratch,
        compiler_params=pltpu.CompilerParams(needs_layout_passes=False),
    )(rank, x1, y1, x2, y2)


@functools.partial(jax.jit, static_argnums=(4, 5))
def _run_nms(x1s, y1s, x2s, y2s, shape3, n_real):
    return pl.pallas_call(
        functools.partial(_nms_body, n_real=n_real),
        out_shape=jax.ShapeDtypeStruct(shape3, jnp.float32),
        scratch_shapes=[
            pltpu.VMEM(shape3, jnp.float32),
            pltpu.VMEM(shape3, jnp.float32),
        ],
    )(x1s, y1s, x2s, y2s)


@jax.jit
def _run_compact(rank, keep_sorted, cls_a, x1, y1, x2, y2, sc_a):
    nb, npad = keep_sorted.shape
    mesh = plsc.VectorSubcoreMesh(core_axis_name="c", subcore_axis_name="s")
    out_type = [jax.ShapeDtypeStruct((nb, _MAX_OUT), jnp.float32)] * 6
    scratch = [pltpu.VMEM((npad,), jnp.int32)] + \
              [pltpu.VMEM((npad,), jnp.float32)] * 7 + \
              [pltpu.VMEM((_MAX_OUT,), jnp.float32)] * 6
    return pl.kernel(
        _compact_body,
        out_type=out_type,
        mesh=mesh,
        scratch_types=scratch,
        compiler_params=pltpu.CompilerParams(needs_layout_passes=False),
    )(rank, keep_sorted, cls_a, x1, y1, x2, y2, sc_a)


def kernel(cls_proposals, gt_classes, box_proposals, gt_boxes, proposal_scores):
    nb = gt_boxes.shape[0]
    cls_all = jnp.concatenate([gt_classes, cls_proposals], axis=1)
    box_all = jnp.concatenate([gt_boxes, box_proposals], axis=1)
    sc_all = jnp.concatenate([gt_classes, proposal_scores], axis=1)
    n = box_all.shape[1]
    npad = ((n + 1023) // 1024) * 1024

    x1 = box_all[:, :, 0]
    y1 = box_all[:, :, 1]
    x2 = box_all[:, :, 2]
    y2 = box_all[:, :, 3]

    def pad_cols(arr, value=0.0):
        return jnp.pad(arr, ((0, 0), (0, npad - n)), constant_values=value)

    s_rows = pad_cols(sc_all, -1.0)
    x1p = pad_cols(x1)
    y1p = pad_cols(y1)
    x2p = pad_cols(x2)
    y2p = pad_cols(y2)

    rank = _run_rank(s_rows, s_rows.T)
    sx1, sy1, sx2, sy2 = _run_permute(rank, x1p, y1p, x2p, y2p)

    fcols = npad // 8
    shape3 = (nb, 8, fcols)
    keep_sorted = _run_nms(sx1.reshape(shape3), sy1.reshape(shape3),
                           sx2.reshape(shape3), sy2.reshape(shape3),
                           shape3, n).reshape(nb, npad)

    oc, o1, o2, o3, o4, osc = _run_compact(
        rank, keep_sorted, pad_cols(cls_all), x1p, y1p, x2p, y2p,
        pad_cols(sc_all))

    outb = jnp.stack([o1, o2, o3, o4], axis=-1)
    return oc, outb, osc


# speedup vs baseline: 1.4557x; 1.4557x over previous
import functools

import jax
import jax.numpy as jnp
from jax import lax
from jax.experimental import pallas as pl
from jax.experimental.pallas import tpu as pltpu
from jax.experimental.pallas import tpu_sc as plsc

_NMS_THR = 0.3
_MAX_OUT = 320
_SPEC_K = 8


def _nms_body(scores_ref, x1_ref, y1_ref, x2_ref, y2_ref, keep_ref, s_ref, a_ref):
    n_b = scores_ref.shape[0]
    rows, fcols = scores_ref.shape[1], scores_ref.shape[2]
    for b in range(n_b):
        x1 = x1_ref[b]
        y1 = y1_ref[b]
        x2 = x2_ref[b]
        y2 = y2_ref[b]
        a_ref[b] = jnp.maximum(x2 - x1, 0.0) * jnp.maximum(y2 - y1, 0.0)
        s_ref[b] = scores_ref[b]
    col = lax.broadcasted_iota(jnp.int32, (rows, fcols), 1)
    row = lax.broadcasted_iota(jnp.int32, (rows, fcols), 0)
    gidx = (col + fcols * row).astype(jnp.float32)
    nbig = jnp.float32(rows * fcols)

    def red(op, arr):
        return op(op(arr, axis=1, keepdims=True), axis=0, keepdims=True)

    def chain(b, m1):
        s = s_ref[b]
        a = a_ref[b]
        x1 = x1_ref[b]
        y1 = y1_ref[b]
        x2 = x2_ref[b]
        y2 = y2_ref[b]

        cands = []
        s_cur = s
        m = m1
        for g in range(_SPEC_K):
            act = m > -0.5
            idx = red(jnp.min, jnp.where(s_cur == m, gidx, nbig))
            cand = (gidx == idx) & act
            cands.append((cand, act))
            s_cur = jnp.where(cand, -3.0, s_cur)
            if g + 1 < _SPEC_K:
                m = red(jnp.max, s_cur)

        coords = []
        for cand, act in cands:
            cf = cand.astype(jnp.float32)
            coords.append((red(jnp.sum, x1 * cf), red(jnp.sum, y1 * cf),
                           red(jnp.sum, x2 * cf), red(jnp.sum, y2 * cf),
                           red(jnp.sum, a * cf)))

        def pair_iou(i, j):
            ix1, iy1, ix2, iy2, ia = coords[i]
            jx1, jy1, jx2, jy2, ja = coords[j]
            w = jnp.maximum(jnp.minimum(ix2, jx2) - jnp.maximum(ix1, jx1), 0.0)
            h = jnp.maximum(jnp.minimum(iy2, jy2) - jnp.maximum(iy1, jy1), 0.0)
            inter = w * h
            return inter / jnp.maximum(ia + ja - inter, 1e-9)

        commit = [cands[0][1]]
        for g in range(1, _SPEC_K):
            sup = commit[0] & (pair_iou(0, g) > _NMS_THR)
            for h in range(1, g):
                sup = sup | (commit[h] & (pair_iou(h, g) > _NMS_THR))
            commit.append(cands[g][1] & jnp.logical_not(sup))

        supp = None
        commit_mask = None
        for g in range(_SPEC_K):
            gx1, gy1, gx2, gy2, ga = coords[g]
            w = jnp.maximum(jnp.minimum(x2, gx2) - jnp.maximum(x1, gx1), 0.0)
            h = jnp.maximum(jnp.minimum(y2, gy2) - jnp.maximum(y1, gy1), 0.0)
            inter = w * h
            iou = inter / jnp.maximum(ga + a - inter, 1e-9)
            sg = (iou > _NMS_THR) & commit[g]
            cg = cands[g][0] & commit[g]
            supp = sg if supp is None else (supp | sg)
            commit_mask = cg if commit_mask is None else (commit_mask | cg)

        s_new = jnp.where(commit_mask, -2.0, jnp.where(supp, -1.0, s))
        s_ref[b] = s_new
        return red(jnp.max, s_new)

    def body(carry):
        return tuple(chain(b, carry[b]) for b in range(n_b))

    def cond(carry):
        m = carry[0]
        for b in range(1, n_b):
            m = jnp.maximum(m, carry[b])
        return m[0, 0] > -0.5

    init = tuple(red(jnp.max, s_ref[b]) for b in range(n_b))
    lax.while_loop(cond, body, init)
    for b in range(n_b):
        keep_ref[b] = (s_ref[b] == -2.0).astype(jnp.float32)


def _compact_body(keep_hbm, cls_hbm, x1_hbm, y1_hbm, x2_hbm, y2_hbm, sc_hbm,
                  oc_hbm, o1_hbm, o2_hbm, o3_hbm, o4_hbm, os_hbm,
                  k_s, c_s, x1_s, y1_s, x2_s, y2_s, s_s,
                  oc_s, o1_s, o2_s, o3_s, o4_s, os_s):
    nb = keep_hbm.shape[0]
    npad = keep_hbm.shape[1]
    wid = lax.axis_index("s") * 2 + lax.axis_index("c")

    @pl.when(wid < nb)
    def _():
        pltpu.sync_copy(keep_hbm.at[wid], k_s)
        pltpu.sync_copy(cls_hbm.at[wid], c_s)
        pltpu.sync_copy(x1_hbm.at[wid], x1_s)
        pltpu.sync_copy(y1_hbm.at[wid], y1_s)
        pltpu.sync_copy(x2_hbm.at[wid], x2_s)
        pltpu.sync_copy(y2_hbm.at[wid], y2_s)
        pltpu.sync_copy(sc_hbm.at[wid], s_s)

        outs = (oc_s, o1_s, o2_s, o3_s, o4_s, os_s)
        srcs = (c_s, x1_s, y1_s, x2_s, y2_s, s_s)

        def zero(i, _):
            z = jnp.zeros((16,), jnp.float32)
            for oref in outs:
                oref[pl.ds(i * 16, 16)] = z
            return 0

        lax.fori_loop(0, _MAX_OUT // 16, zero, 0)

        def step(i, base):
            kv = k_s[pl.ds(i * 16, 16)]
            ci = plsc.cumsum(kv)
            pos = base + ci.astype(jnp.int32) - 1
            msk = (kv > 0.5) & (pos < _MAX_OUT)
            for src, dst in zip(srcs, outs):
                plsc.store_scatter(dst, [pos], src[pl.ds(i * 16, 16)], mask=msk)
            return base + jnp.sum(kv).astype(jnp.int32)

        lax.fori_loop(0, npad // 16, step, jnp.int32(0))

        pltpu.sync_copy(oc_s, oc_hbm.at[wid])
        pltpu.sync_copy(o1_s, o1_hbm.at[wid])
        pltpu.sync_copy(o2_s, o2_hbm.at[wid])
        pltpu.sync_copy(o3_s, o3_hbm.at[wid])
        pltpu.sync_copy(o4_s, o4_hbm.at[wid])
        pltpu.sync_copy(os_s, os_hbm.at[wid])


@functools.partial(jax.jit, static_argnums=(5,))
def _run_nms(scores8, x18, y18, x28, y28, shape3):
    return pl.pallas_call(
        _nms_body,
        out_shape=jax.ShapeDtypeStruct(shape3, jnp.float32),
        scratch_shapes=[
            pltpu.VMEM(shape3, jnp.float32),
            pltpu.VMEM(shape3, jnp.float32),
        ],
    )(scores8, x18, y18, x28, y28)


@functools.partial(jax.jit, static_argnums=(7, 8))
def _run_compact(keep, cls_a, x1, y1, x2, y2, sc_a, nb, npad):
    mesh = plsc.VectorSubcoreMesh(core_axis_name="c", subcore_axis_name="s")
    out_type = [jax.ShapeDtypeStruct((nb, _MAX_OUT), jnp.float32)] * 6
    scratch = [pltpu.VMEM((npad,), jnp.float32)] * 7 + \
              [pltpu.VMEM((_MAX_OUT,), jnp.float32)] * 6
    return pl.kernel(
        _compact_body,
        out_type=out_type,
        mesh=mesh,
        scratch_types=scratch,
        compiler_params=pltpu.CompilerParams(needs_layout_passes=False),
    )(keep, cls_a, x1, y1, x2, y2, sc_a)


def kernel(cls_proposals, gt_classes, box_proposals, gt_boxes, proposal_scores):
    nb = gt_boxes.shape[0]
    cls_all = jnp.concatenate([gt_classes, cls_proposals], axis=1)
    box_all = jnp.concatenate([gt_boxes, box_proposals], axis=1)
    sc_all = jnp.concatenate([gt_classes, proposal_scores], axis=1)
    n = box_all.shape[1]
    npad = ((n + 511) // 512) * 512

    x1 = box_all[:, :, 0]
    y1 = box_all[:, :, 1]
    x2 = box_all[:, :, 2]
    y2 = box_all[:, :, 3]

    fcols = npad // 8
    shape3 = (nb, 8, fcols)

    def fold(arr, value):
        out = jnp.full((nb, npad), value, jnp.float32)
        out = out.at[:, :n].set(arr)
        return out.reshape(shape3)

    scores8 = fold(sc_all, -1.0)
    x18 = fold(x1, 0.0)
    y18 = fold(y1, 0.0)
    x28 = fold(x2, 0.0)
    y28 = fold(y2, 0.0)

    keep = _run_nms(scores8, x18, y18, x28, y28, shape3).reshape(nb, npad)

    def pad_cols(arr):
        return jnp.pad(arr, ((0, 0), (0, npad - n)))

    oc, o1, o2, o3, o4, osc = _run_compact(
        keep, pad_cols(cls_all), pad_cols(x1), pad_cols(y1), pad_cols(x2),
        pad_cols(y2), pad_cols(sc_all), nb, npad)

    outb = jnp.stack([o1, o2, o3, o4], axis=-1)
    return oc, outb, osc
